# baseline (speedup 1.0000x reference)
import functools

import jax
import jax.numpy as jnp
from jax import lax
from jax.experimental import pallas as pl
from jax.experimental.pallas import tpu as pltpu
from jax.experimental.pallas import tpu_sc as plsc

_B, _L = 16384, 200
_TOTAL = _B * _L
_NC, _NS = 2, 16
_NW = _NC * _NS
_PER_W = _TOTAL // _NW
_CHUNK = 12800
_NCHUNKS = _PER_W // _CHUNK
_UNROLL = 8
_TBL_PAD = 40
_LANES = 16

_mesh = plsc.VectorSubcoreMesh(core_axis_name="c", subcore_axis_name="s")


@functools.partial(
    pl.kernel,
    mesh=_mesh,
    out_type=jax.ShapeDtypeStruct((_TOTAL,), jnp.float32),
    scratch_types=[
        pltpu.VMEM((_TBL_PAD,), jnp.float32),
        pltpu.VMEM((_CHUNK,), jnp.int32),
        pltpu.VMEM((_CHUNK,), jnp.float32),
    ],
    compiler_params=pltpu.CompilerParams(needs_layout_passes=False),
)
def _lut_gather(idx_hbm, tbl_hbm, out_hbm, tbl_v, idx_v, out_v):
    wid = lax.axis_index("s") * _NC + lax.axis_index("c")
    base = wid * _PER_W
    pltpu.sync_copy(tbl_hbm, tbl_v)

    def chunk_body(ci, carry):
        off = base + ci * _CHUNK
        pltpu.sync_copy(idx_hbm.at[pl.ds(off, _CHUNK)], idx_v)

        def vec_body(g, c):
            for u in range(_UNROLL):
                s = g * (_LANES * _UNROLL) + u * _LANES
                ids = idx_v[pl.ds(s, _LANES)]
                ids = jnp.minimum(jnp.maximum(ids, 0), 34)
                out_v[pl.ds(s, _LANES)] = plsc.load_gather(tbl_v, [ids])
            return c

        lax.fori_loop(0, _CHUNK // (_LANES * _UNROLL), vec_body, 0)
        pltpu.sync_copy(out_v, out_hbm.at[pl.ds(off, _CHUNK)])
        return carry

    lax.fori_loop(0, _NCHUNKS, chunk_body, 0)


def kernel(discrete_f0, table):
    flat_idx = discrete_f0.reshape(_TOTAL).astype(jnp.int32)
    tbl = jnp.pad(table.reshape(-1).astype(jnp.float32),
                  (0, _TBL_PAD - table.shape[0]))
    out = _lut_gather(flat_idx, tbl)
    return out.reshape(_B, _L)


# speedup vs baseline: 1.0312x; 1.0312x over previous
import functools

import jax
import jax.numpy as jnp
from jax import lax
from jax.experimental import pallas as pl
from jax.experimental.pallas import tpu as pltpu
from jax.experimental.pallas import tpu_sc as plsc

_B, _L = 16384, 200
_TOTAL = _B * _L
_NC, _NS = 2, 16
_NW = _NC * _NS
_PER_W = _TOTAL // _NW
_CHUNK = 12800
_NCHUNKS = _PER_W // _CHUNK
_UNROLL = 8
_TBL_PAD = 40
_LANES = 16

_mesh = plsc.VectorSubcoreMesh(core_axis_name="c", subcore_axis_name="s")


@functools.partial(
    pl.kernel,
    mesh=_mesh,
    out_type=jax.ShapeDtypeStruct((_TOTAL,), jnp.float32),
    scratch_types=[
        pltpu.VMEM((_TBL_PAD,), jnp.float32),
        pltpu.VMEM((2, _CHUNK), jnp.int32),
        pltpu.VMEM((2, _CHUNK), jnp.float32),
        pltpu.SemaphoreType.DMA,
        pltpu.SemaphoreType.DMA,
        pltpu.SemaphoreType.DMA,
        pltpu.SemaphoreType.DMA,
    ],
    compiler_params=pltpu.CompilerParams(needs_layout_passes=False),
)
def _lut_gather(idx_hbm, tbl_hbm, out_hbm, tbl_v, idx_v, out_v,
                sin0, sin1, sout0, sout1):
    wid = lax.axis_index("s") * _NC + lax.axis_index("c")
    base = wid * _PER_W
    sins = (sin0, sin1)
    souts = (sout0, sout1)
    pltpu.sync_copy(tbl_hbm, tbl_v)

    def start_in(ci):
        off = base + ci * _CHUNK
        return pltpu.async_copy(idx_hbm.at[pl.ds(off, _CHUNK)],
                                idx_v.at[ci % 2], sins[ci % 2])

    in_handles = [None, None]
    out_handles = [None, None]
    in_handles[0] = start_in(0)

    for ci in range(_NCHUNKS):
        slot = ci % 2
        in_handles[slot].wait()
        if ci + 1 < _NCHUNKS:
            in_handles[(ci + 1) % 2] = start_in(ci + 1)
        if out_handles[slot] is not None:
            out_handles[slot].wait()

        def vec_body(g, c, slot=slot):
            for u in range(_UNROLL):
                s = g * (_LANES * _UNROLL) + u * _LANES
                ids = idx_v[slot, pl.ds(s, _LANES)]
                ids = jnp.minimum(jnp.maximum(ids, 0), 34)
                out_v[slot, pl.ds(s, _LANES)] = plsc.load_gather(tbl_v, [ids])
            return c

        lax.fori_loop(0, _CHUNK // (_LANES * _UNROLL), vec_body, 0)
        out_handles[slot] = pltpu.async_copy(
            out_v.at[slot],
            out_hbm.at[pl.ds(base + ci * _CHUNK, _CHUNK)], souts[slot])

    for h in out_handles:
        if h is not None:
            h.wait()


def kernel(discrete_f0, table):
    flat_idx = discrete_f0.reshape(_TOTAL).astype(jnp.int32)
    tbl = jnp.pad(table.reshape(-1).astype(jnp.float32),
                  (0, _TBL_PAD - table.shape[0]))
    out = _lut_gather(flat_idx, tbl)
    return out.reshape(_B, _L)


# speedup vs baseline: 1.7644x; 1.7111x over previous
import functools

import jax
import jax.numpy as jnp
from jax import lax
from jax.experimental import pallas as pl
from jax.experimental.pallas import tpu as pltpu
from jax.experimental.pallas import tpu_sc as plsc

_B, _L = 16384, 200
_NC, _NS = 2, 16
_NW = _NC * _NS
_ROWS_W = _B // _NW
_CHUNK_R = 64
_NCHUNKS = _ROWS_W // _CHUNK_R
_TBL_PAD = 40
_LANES = 16
_COL_OFFS = tuple(range(0, 192, 16)) + (184,)

_mesh = plsc.VectorSubcoreMesh(core_axis_name="c", subcore_axis_name="s")


@functools.partial(
    pl.kernel,
    mesh=_mesh,
    out_type=jax.ShapeDtypeStruct((_B, _L), jnp.float32),
    scratch_types=[
        pltpu.VMEM((_TBL_PAD,), jnp.float32),
        pltpu.VMEM((2, _CHUNK_R, _L), jnp.int32),
        pltpu.VMEM((2, _CHUNK_R, _L), jnp.float32),
        pltpu.SemaphoreType.DMA,
        pltpu.SemaphoreType.DMA,
        pltpu.SemaphoreType.DMA,
        pltpu.SemaphoreType.DMA,
    ],
    compiler_params=pltpu.CompilerParams(
        needs_layout_passes=False, use_tc_tiling_on_sc=True),
)
def _lut_gather(idx_hbm, tbl_hbm, out_hbm, tbl_v, idx_v, out_v,
                sin0, sin1, sout0, sout1):
    wid = lax.axis_index("s") * _NC + lax.axis_index("c")
    base = wid * _ROWS_W
    sins = (sin0, sin1)
    souts = (sout0, sout1)
    pltpu.sync_copy(tbl_hbm, tbl_v)

    def start_in(ci):
        off = base + ci * _CHUNK_R
        return pltpu.async_copy(idx_hbm.at[pl.ds(off, _CHUNK_R), :],
                                idx_v.at[ci % 2], sins[ci % 2])

    in_handles = [None, None]
    out_handles = [None, None]
    in_handles[0] = start_in(0)

    for ci in range(_NCHUNKS):
        slot = ci % 2
        in_handles[slot].wait()
        if ci + 1 < _NCHUNKS:
            in_handles[(ci + 1) % 2] = start_in(ci + 1)
        if out_handles[slot] is not None:
            out_handles[slot].wait()

        def row_body(r, c, slot=slot):
            for co in _COL_OFFS:
                ids = idx_v[slot, r, pl.ds(co, _LANES)]
                ids = jnp.minimum(jnp.maximum(ids, 0), 34)
                out_v[slot, r, pl.ds(co, _LANES)] = plsc.load_gather(
                    tbl_v, [ids])
            return c

        lax.fori_loop(0, _CHUNK_R, row_body, 0)
        out_handles[slot] = pltpu.async_copy(
            out_v.at[slot],
            out_hbm.at[pl.ds(base + ci * _CHUNK_R, _CHUNK_R), :], souts[slot])

    for h in out_handles:
        if h is not None:
            h.wait()


def kernel(discrete_f0, table):
    idx = discrete_f0.astype(jnp.int32)
    tbl = jnp.pad(table.reshape(-1).astype(jnp.float32),
                  (0, _TBL_PAD - table.shape[0]))
    return _lut_gather(idx, tbl)
